# baseline (speedup 1.0000x reference)
import functools

import jax
import jax.numpy as jnp
from jax import lax
from jax.experimental import pallas as pl
from jax.experimental.pallas import tpu as pltpu
from jax.experimental.pallas import tpu_sc as plsc

NC = 2
NS = 16
L = 16
NW = NC * NS

EMB = 64
ROW = 72
CHUNK = 128


def _make_sc_kernel(batch: int):
    bpw = batch // NW
    nchunk = bpw // CHUNK
    blk_per_chunk = CHUNK // L

    mesh = plsc.VectorSubcoreMesh(core_axis_name="c", subcore_axis_name="s")

    @functools.partial(
        pl.kernel,
        mesh=mesh,
        compiler_params=pltpu.CompilerParams(
            needs_layout_passes=False, use_tc_tiling_on_sc=False),
        out_type=jax.ShapeDtypeStruct((batch,), jnp.float32),
        scratch_types=[
            pltpu.VMEM((nchunk, CHUNK), jnp.int32),
            pltpu.VMEM((nchunk, CHUNK), jnp.int32),
            pltpu.VMEM((bpw, ROW), jnp.float32),
            pltpu.VMEM((bpw, ROW), jnp.float32),
            pltpu.VMEM((L,), jnp.float32),
            pltpu.VMEM((bpw,), jnp.float32),
            pltpu.SemaphoreType.DMA,
            pltpu.SemaphoreType.DMA,
            pltpu.SemaphoreType.DMA,
            pltpu.SemaphoreType.DMA,
        ],
    )
    def sc_kernel(uid_hbm, iid_hbm, ucat_hbm, icat_hbm, mean_hbm, out_hbm,
                  uid_v, iid_v, urows_v, irows_v, mean_v, out_v,
                  sem0, sem1, sem2, sem3):
        sems = [sem0, sem1, sem2, sem3]
        wid = lax.axis_index("s") * NC + lax.axis_index("c")

        pltpu.sync_copy(uid_hbm.at[wid], uid_v)
        pltpu.sync_copy(iid_hbm.at[wid], iid_v)
        pltpu.sync_copy(mean_hbm, mean_v)

        copies = []
        for j in range(nchunk):
            dst = pl.ds(j * CHUNK, CHUNK)
            cu = pltpu.async_copy(ucat_hbm.at[uid_v.at[j]], urows_v.at[dst], sems[j])
            ci = pltpu.async_copy(icat_hbm.at[iid_v.at[j]], irows_v.at[dst], sems[j])
            copies.append((cu, ci))

        mean_vec = mean_v[...]
        lanes = lax.iota(jnp.int32, L)
        bias_col = jnp.full((L,), EMB, jnp.int32)

        def blk_body(blk, carry):
            base = blk * L
            row = base + lanes
            accs = [
                mean_vec
                + plsc.load_gather(urows_v, [row, bias_col])
                + plsc.load_gather(irows_v, [row, bias_col]),
                jnp.zeros((L,), jnp.float32),
                jnp.zeros((L,), jnp.float32),
                jnp.zeros((L,), jnp.float32),
            ]
            for d in range(EMB):
                col = jnp.full((L,), d, jnp.int32)
                u = plsc.load_gather(urows_v, [row, col])
                iv = plsc.load_gather(irows_v, [row, col])
                accs[d % 4] = accs[d % 4] + u * iv
            out_v[pl.ds(base, L)] = (accs[0] + accs[1]) + (accs[2] + accs[3])
            return carry

        for j in range(nchunk):
            copies[j][0].wait()
            copies[j][1].wait()
            lax.fori_loop(j * blk_per_chunk, (j + 1) * blk_per_chunk,
                          blk_body, 0)

        pltpu.sync_copy(out_v, out_hbm.at[pl.ds(wid * bpw, bpw)])

    return sc_kernel


def kernel(u_id, i_id, user_emb, user_bias, item_emb, item_bias,
           user_visual_emb, mean):
    batch = u_id.shape[0]
    uid3 = u_id.reshape(NW, batch // NW // CHUNK, CHUNK)
    iid3 = i_id.reshape(NW, batch // NW // CHUNK, CHUNK)
    pad = jnp.zeros((user_emb.shape[0], ROW - EMB - 1), jnp.float32)
    ucat = jnp.concatenate([user_emb, user_bias, pad], axis=1)
    icat = jnp.concatenate([item_emb, item_bias, pad], axis=1)
    mean_l = jnp.broadcast_to(mean, (L,))
    sc = _make_sc_kernel(batch)
    return sc(uid3, iid3, ucat, icat, mean_l)


# speedup vs baseline: 1.8702x; 1.8702x over previous
import functools

import jax
import jax.numpy as jnp
from jax import lax
from jax.experimental import pallas as pl
from jax.experimental.pallas import tpu as pltpu
from jax.experimental.pallas import tpu_sc as plsc

NC = 2
NS = 16
L = 16
NW = NC * NS

EMB = 64
CHUNK = 128
BIAS_PAD = 102400
BIAS_SLICE = BIAS_PAD // NS


def _make_sc_kernel(batch: int):
    bpw = batch // NW
    nchunk = bpw // CHUNK
    blk_per_chunk = CHUNK // L

    mesh = plsc.VectorSubcoreMesh(core_axis_name="c", subcore_axis_name="s")

    @functools.partial(
        pl.kernel,
        mesh=mesh,
        compiler_params=pltpu.CompilerParams(
            needs_layout_passes=False, use_tc_tiling_on_sc=False),
        out_type=jax.ShapeDtypeStruct((batch,), jnp.float32),
        scratch_types=[
            pltpu.VMEM((nchunk, CHUNK), jnp.int32),
            pltpu.VMEM((nchunk, CHUNK), jnp.int32),
            pltpu.VMEM((bpw, EMB), jnp.float32),
            pltpu.VMEM((bpw, EMB), jnp.float32),
            pltpu.VMEM((bpw,), jnp.float32),
            pltpu.VMEM((bpw,), jnp.float32),
            pltpu.VMEM((L,), jnp.float32),
            pltpu.VMEM((bpw,), jnp.float32),
            pltpu.VMEM_SHARED((BIAS_PAD,), jnp.float32),
            pltpu.VMEM_SHARED((BIAS_PAD,), jnp.float32),
            pltpu.SemaphoreType.DMA,
            pltpu.SemaphoreType.DMA,
            pltpu.SemaphoreType.DMA,
            pltpu.SemaphoreType.DMA,
            pltpu.SemaphoreType.DMA,
        ],
    )
    def sc_kernel(uid_hbm, iid_hbm, uemb_hbm, iemb_hbm, ubias_hbm, ibias_hbm,
                  mean_hbm, out_hbm,
                  uid_v, iid_v, urows_v, irows_v, ubias_v, ibias_v, mean_v,
                  out_v, ubias_sp, ibias_sp, sem0, sem1, sem2, sem3, semb):
        sems = [sem0, sem1, sem2, sem3]
        cid = lax.axis_index("c")
        sid = lax.axis_index("s")
        wid = sid * NC + cid

        pltpu.sync_copy(uid_hbm.at[wid], uid_v)
        pltpu.sync_copy(iid_hbm.at[wid], iid_v)
        pltpu.sync_copy(mean_hbm, mean_v)

        copies = []
        for j in range(nchunk):
            dst = pl.ds(j * CHUNK, CHUNK)
            cu = pltpu.async_copy(uemb_hbm.at[uid_v.at[j]], urows_v.at[dst], sems[j])
            ci = pltpu.async_copy(iemb_hbm.at[iid_v.at[j]], irows_v.at[dst], sems[j])
            copies.append((cu, ci))

        bsl = pl.ds(sid * BIAS_SLICE, BIAS_SLICE)
        pltpu.sync_copy(ubias_hbm.at[bsl], ubias_sp.at[bsl])
        pltpu.sync_copy(ibias_hbm.at[bsl], ibias_sp.at[bsl])
        plsc.subcore_barrier()
        bias_copies = []
        for j in range(nchunk):
            dst = pl.ds(j * CHUNK, CHUNK)
            bias_copies.append(
                pltpu.async_copy(ubias_sp.at[uid_v.at[j]], ubias_v.at[dst], semb))
            bias_copies.append(
                pltpu.async_copy(ibias_sp.at[iid_v.at[j]], ibias_v.at[dst], semb))
        for c in bias_copies:
            c.wait()

        mean_vec = mean_v[...]
        lanes = lax.iota(jnp.int32, L)

        def blk_body(blk, carry):
            base = blk * L
            row = base + lanes
            accs = [
                mean_vec + ubias_v[pl.ds(base, L)] + ibias_v[pl.ds(base, L)],
                jnp.zeros((L,), jnp.float32),
                jnp.zeros((L,), jnp.float32),
                jnp.zeros((L,), jnp.float32),
            ]
            for d in range(EMB):
                col = jnp.full((L,), d, jnp.int32)
                u = plsc.load_gather(urows_v, [row, col])
                iv = plsc.load_gather(irows_v, [row, col])
                accs[d % 4] = accs[d % 4] + u * iv
            out_v[pl.ds(base, L)] = (accs[0] + accs[1]) + (accs[2] + accs[3])
            return carry

        for j in range(nchunk):
            copies[j][0].wait()
            copies[j][1].wait()
            lax.fori_loop(j * blk_per_chunk, (j + 1) * blk_per_chunk,
                          blk_body, 0)

        pltpu.sync_copy(out_v, out_hbm.at[pl.ds(wid * bpw, bpw)])

    return sc_kernel


def kernel(u_id, i_id, user_emb, user_bias, item_emb, item_bias,
           user_visual_emb, mean):
    batch = u_id.shape[0]
    uid3 = u_id.reshape(NW, batch // NW // CHUNK, CHUNK)
    iid3 = i_id.reshape(NW, batch // NW // CHUNK, CHUNK)
    nu = user_bias.shape[0]
    ub = jnp.pad(user_bias.reshape(-1), (0, BIAS_PAD - nu))
    ib = jnp.pad(item_bias.reshape(-1), (0, BIAS_PAD - nu))
    mean_l = jnp.broadcast_to(mean, (L,))
    sc = _make_sc_kernel(batch)
    return sc(uid3, iid3, user_emb, item_emb, ub, ib, mean_l)
